# baseline (speedup 1.0000x reference)
import functools

import jax
import jax.numpy as jnp
from jax import lax
from jax.experimental import pallas as pl
from jax.experimental.pallas import tpu as pltpu
from jax.experimental.pallas import tpu_sc as plsc

BATCH = 16384

_info = plsc.get_sparse_core_info()
_NC, _NS = 1, _info.num_subcores
_NW = _NC * _NS
_BPW = BATCH // _NW
_CHUNK = 256
_NCHUNK = _BPW // _CHUNK

_mesh = plsc.VectorSubcoreMesh(core_axis_name="c", subcore_axis_name="s",
                               num_cores=_NC, num_subcores=_NS)


@functools.partial(
    pl.kernel,
    mesh=_mesh,
    out_type=jax.ShapeDtypeStruct((BATCH,), jnp.int32),
    scratch_types=[
        pltpu.VMEM((_BPW,), jnp.int32),
        pltpu.VMEM((_BPW,), jnp.int32),
    ] + [pltpu.SemaphoreType.DMA] * (2 * _NCHUNK + 1),
)
def _gather_kernel(idx_hbm, table_hbm, out_hbm, idx_v, vals_v, *sems):
    ssems = sems[:_NCHUNK]
    gsems, sem_s = sems[_NCHUNK:2 * _NCHUNK], sems[2 * _NCHUNK]
    wid = lax.axis_index("s") * _NC + lax.axis_index("c")
    base = wid * _BPW
    stages = []
    for j in range(_NCHUNK):
        sl = pl.ds(j * _CHUNK, _CHUNK)
        stages.append(
            pltpu.async_copy(idx_hbm.at[pl.ds(base + j * _CHUNK, _CHUNK)],
                             idx_v.at[sl], ssems[j])
        )
    gathers = []
    for j in range(_NCHUNK):
        sl = pl.ds(j * _CHUNK, _CHUNK)
        stages[j].wait()
        gathers.append(
            pltpu.async_copy(table_hbm.at[idx_v.at[sl]], vals_v.at[sl], gsems[j])
        )
    stores = []
    for j in range(_NCHUNK):
        sl = pl.ds(j * _CHUNK, _CHUNK)
        gathers[j].wait()
        stores.append(
            pltpu.async_copy(vals_v.at[sl],
                             out_hbm.at[pl.ds(base + j * _CHUNK, _CHUNK)], sem_s)
        )
    for c in stores:
        c.wait()


def kernel(idx, lookup_table):
    return _gather_kernel(idx, lookup_table)


# speedup vs baseline: 1.0038x; 1.0038x over previous
import functools

import jax
import jax.numpy as jnp
from jax import lax
from jax.experimental import pallas as pl
from jax.experimental.pallas import tpu as pltpu
from jax.experimental.pallas import tpu_sc as plsc

BATCH = 16384

_info = plsc.get_sparse_core_info()
_NC, _NS = 1, _info.num_subcores
_NW = _NC * _NS
_BPW = BATCH // _NW
_CHUNK = 256
_NCHUNK = _BPW // _CHUNK

_mesh = plsc.VectorSubcoreMesh(core_axis_name="c", subcore_axis_name="s",
                               num_cores=_NC, num_subcores=_NS)


@functools.partial(
    pl.kernel,
    mesh=_mesh,
    out_type=jax.ShapeDtypeStruct((BATCH,), jnp.int32),
    scratch_types=[
        pltpu.VMEM((_BPW,), jnp.int32),
        pltpu.VMEM((_BPW,), jnp.int32),
    ] + [pltpu.SemaphoreType.DMA] * (_NCHUNK + 1),
)
def _gather_kernel(idx_hbm, table_hbm, out_hbm, idx_v, vals_v, *sems):
    gsems, sem_s = sems[:_NCHUNK], sems[_NCHUNK]
    wid = lax.axis_index("s") * _NC + lax.axis_index("c")
    base = wid * _BPW
    pltpu.sync_copy(idx_hbm.at[pl.ds(base, _BPW)], idx_v)
    gathers = []
    for j in range(_NCHUNK):
        sl = pl.ds(j * _CHUNK, _CHUNK)
        gathers.append(
            pltpu.async_copy(table_hbm.at[idx_v.at[sl]], vals_v.at[sl], gsems[j])
        )
    stores = []
    for j in range(_NCHUNK):
        sl = pl.ds(j * _CHUNK, _CHUNK)
        gathers[j].wait()
        stores.append(
            pltpu.async_copy(vals_v.at[sl],
                             out_hbm.at[pl.ds(base + j * _CHUNK, _CHUNK)], sem_s)
        )
    for c in stores:
        c.wait()


def kernel(idx, lookup_table):
    return _gather_kernel(idx, lookup_table)
